# Initial kernel scaffold; baseline (speedup 1.0000x reference)
import jax
import jax.numpy as jnp
from jax.experimental import pallas as pl


def kernel(x, edge_index, batch, Wk0, bk0, Wq0, bq0, Wv0, bv0, Ws0, bs0, gnw0, gnb0, gnm0, Wk1, bk1, Wq1, bq1, Wv1, bv1, Ws1, bs1, gnw1, gnb1, gnm1, Wh0, bh0, lnw0, lnb0, Wh1, bh1, lnw1, lnb1, Wl, bl):
    raise NotImplementedError("write your pallas kernel here")


# speedup vs baseline: 5.4416x; 5.4416x over previous
import functools

import jax
import jax.numpy as jnp
from jax import lax
from jax.experimental import pallas as pl
from jax.experimental.pallas import tpu as pltpu
from jax.experimental.pallas import tpu_sc as plsc

N_NODES = 10000
N_EDGES = 320000
G = 64
D = 128
H1 = 128
H2 = 64
NCLS = 8

NW = 32
EPT = N_EDGES // NW
CHK = 80
NCHUNK = EPT // CHK
N_PAD = 10240
ROWS_PT = N_PAD // 16

BLK = 1000
NBLK = N_NODES // BLK

_HI = lax.Precision.HIGHEST
_C00 = (((0,), (0,)), ((), ()))


def _mm(a, b):
    return jnp.dot(a, b, preferred_element_type=jnp.float32, precision=_HI)


def _gelu(x):
    return x * 0.5 * (1.0 + lax.erf(x * (2.0 ** -0.5)))


def _proj4_body(h_ref, wk, bk, wq, bq, wv, bv, ws, bs, ko, qo, vo, so):
    h = h_ref[...]
    ko[...] = _mm(h, wk[...]) + bk[...]
    qo[...] = _mm(h, wq[...]) + bq[...]
    vo[...] = _mm(h, wv[...]) + bv[...]
    so[...] = _mm(h, ws[...]) + bs[...]


def _proj4(h, Wk, bk, Wq, bq, Wv, bv, Ws, bs):
    blk = 2000
    row = pl.BlockSpec((blk, D), lambda i: (i, 0))
    wsp = pl.BlockSpec((D, D), lambda i: (0, 0))
    bsp = pl.BlockSpec((1, D), lambda i: (0, 0))
    return pl.pallas_call(
        _proj4_body,
        grid=(N_NODES // blk,),
        in_specs=[row, wsp, bsp, wsp, bsp, wsp, bsp, wsp, bsp],
        out_specs=[row, row, row, row],
        out_shape=[jax.ShapeDtypeStruct((N_NODES, D), jnp.float32)] * 4,
    )(h, Wk, bk.reshape(1, D), Wq, bq.reshape(1, D),
      Wv, bv.reshape(1, D), Ws, bs.reshape(1, D))


def _edge_body(k_hbm, q_hbm, v_hbm, src_hbm, dst_hbm, out_hbm,
               idx_s, idx_d, krows, qrows, vrows, zbuf, acc, sem):
    c = lax.axis_index("c")
    s = lax.axis_index("s")
    wid = c * 16 + s

    def _zrow(r, carry):
        for j in range(D // 16):
            zbuf[r, pl.ds(j * 16, 16)] = jnp.zeros((16,), jnp.float32)
        return carry
    lax.fori_loop(0, 128, _zrow, 0)
    for t in range(ROWS_PT // 128):
        pltpu.sync_copy(zbuf, acc.at[pl.ds(s * ROWS_PT + t * 128, 128)])
    plsc.subcore_barrier()

    def _chunk(i, carry):
        off = wid * EPT + i * CHK
        pltpu.sync_copy(src_hbm.at[pl.ds(off, CHK)], idx_s)
        pltpu.sync_copy(dst_hbm.at[pl.ds(off, CHK)], idx_d)
        ck = pltpu.async_copy(k_hbm.at[idx_d], krows, sem)
        cq = pltpu.async_copy(q_hbm.at[idx_s], qrows, sem)
        cv = pltpu.async_copy(v_hbm.at[idx_s], vrows, sem)
        ck.wait()
        cq.wait()
        cv.wait()

        def _edge(e, ecarry):
            for j in range(D // 16):
                sl = pl.ds(j * 16, 16)
                kk = krows[e, sl]
                qq = qrows[e, sl]
                vv = vrows[e, sl]
                vrows[e, sl] = vv / (1.0 + jnp.exp(-(kk + qq)))
            return ecarry
        lax.fori_loop(0, CHK, _edge, 0)
        pltpu.sync_copy(vrows, acc.at[idx_d], add=True)
        return carry
    lax.fori_loop(0, NCHUNK, _chunk, 0)

    plsc.subcore_barrier()
    for t in range(ROWS_PT // 128):
        r0 = s * ROWS_PT + t * 128
        pltpu.sync_copy(acc.at[pl.ds(r0, 128)], zbuf)
        pltpu.sync_copy(zbuf, out_hbm.at[c, pl.ds(r0, 128)])


_edge_agg = functools.partial(
    pl.kernel,
    out_type=jax.ShapeDtypeStruct((2, N_PAD, D), jnp.float32),
    mesh=plsc.VectorSubcoreMesh(core_axis_name="c", subcore_axis_name="s"),
    scratch_types=[
        pltpu.VMEM((CHK,), jnp.int32),
        pltpu.VMEM((CHK,), jnp.int32),
        pltpu.VMEM((CHK, D), jnp.float32),
        pltpu.VMEM((CHK, D), jnp.float32),
        pltpu.VMEM((CHK, D), jnp.float32),
        pltpu.VMEM((128, D), jnp.float32),
        pltpu.VMEM_SHARED((N_PAD, D), jnp.float32),
        pltpu.SemaphoreType.DMA,
    ],
)(_edge_body)


def _onehot(b2d, rows):
    P = (b2d[...] == lax.broadcasted_iota(jnp.int32, (rows, G), 1))
    return P.astype(jnp.float32)


def _stats_accum(i, P, hg, s1, s2, cnt):
    @pl.when(i == 0)
    def _():
        s1[...] = jnp.zeros((G, D), jnp.float32)
        s2[...] = jnp.zeros((G, D), jnp.float32)
        cnt[...] = jnp.zeros((G, D), jnp.float32)
    s1[...] += lax.dot_general(P, hg, _C00, precision=_HI)
    s2[...] += lax.dot_general(P, hg * hg, _C00, precision=_HI)
    cnt[...] += lax.dot_general(P, jnp.ones_like(hg), _C00, precision=_HI)


def _stats0_body(p0, p1, sk, b2d, hg_out, s1, s2, cnt):
    i = pl.program_id(0)
    hg = _gelu(p0[0] + p1[0] + sk[...])
    hg_out[...] = hg
    _stats_accum(i, _onehot(b2d, BLK), hg, s1, s2, cnt)


def _stats1_body(p0, p1, sk, b2d, s1, s2, cnt):
    i = pl.program_id(0)
    hg = _gelu(p0[0] + p1[0] + sk[...])
    _stats_accum(i, _onehot(b2d, BLK), hg, s1, s2, cnt)


def _stats_specs():
    prow = lambda core: pl.BlockSpec((1, BLK, D), lambda i, c=core: (c, i, 0))
    row = pl.BlockSpec((BLK, D), lambda i: (i, 0))
    bsp = pl.BlockSpec((BLK, 1), lambda i: (i, 0))
    gsp = pl.BlockSpec((G, D), lambda i: (0, 0))
    return prow, row, bsp, gsp


def _stats0(p, sk, b2d):
    prow, row, bsp, gsp = _stats_specs()
    return pl.pallas_call(
        _stats0_body,
        grid=(NBLK,),
        in_specs=[prow(0), prow(1), row, bsp],
        out_specs=[row, gsp, gsp, gsp],
        out_shape=[jax.ShapeDtypeStruct((N_NODES, D), jnp.float32)]
        + [jax.ShapeDtypeStruct((G, D), jnp.float32)] * 3,
    )(p, p, sk, b2d)


def _stats1(p, sk, b2d):
    prow, row, bsp, gsp = _stats_specs()
    return pl.pallas_call(
        _stats1_body,
        grid=(NBLK,),
        in_specs=[prow(0), prow(1), row, bsp],
        out_specs=[gsp, gsp, gsp],
        out_shape=[jax.ShapeDtypeStruct((G, D), jnp.float32)] * 3,
    )(p, p, sk, b2d)


def _gn_moments(s1, s2, cnt, gnm):
    c = jnp.maximum(cnt, 1.0)
    mg = s1 / c * gnm
    var = (s2 - 2.0 * mg * s1 + c * mg * mg) / c
    return mg, var, c


def _normproj_body(hg, s1, s2, cnt, b2d, gnw, gnb, gnm,
                   wk, bk, wq, bq, wv, bv, ws, bs, ko, qo, vo, so):
    mg, var, _ = _gn_moments(s1[...], s2[...], cnt[...], gnm[...])
    P = _onehot(b2d, BLK)
    o = hg[...] - _mm(P, mg)
    h = o * lax.rsqrt(_mm(P, var) + 1e-5) * gnw[...] + gnb[...]
    ko[...] = _mm(h, wk[...]) + bk[...]
    qo[...] = _mm(h, wq[...]) + bq[...]
    vo[...] = _mm(h, wv[...]) + bv[...]
    so[...] = _mm(h, ws[...]) + bs[...]


def _normproj(hg, s1, s2, cnt, b2d, gnw, gnb, gnm,
              Wk, bk, Wq, bq, Wv, bv, Ws, bs):
    row = pl.BlockSpec((BLK, D), lambda i: (i, 0))
    bsp = pl.BlockSpec((BLK, 1), lambda i: (i, 0))
    gsp = pl.BlockSpec((G, D), lambda i: (0, 0))
    vsp = pl.BlockSpec((1, D), lambda i: (0, 0))
    wsp = pl.BlockSpec((D, D), lambda i: (0, 0))
    return pl.pallas_call(
        _normproj_body,
        grid=(NBLK,),
        in_specs=[row, gsp, gsp, gsp, bsp, vsp, vsp, vsp,
                  wsp, vsp, wsp, vsp, wsp, vsp, wsp, vsp],
        out_specs=[row, row, row, row],
        out_shape=[jax.ShapeDtypeStruct((N_NODES, D), jnp.float32)] * 4,
    )(hg, s1, s2, cnt, b2d, gnw.reshape(1, D), gnb.reshape(1, D),
      gnm.reshape(1, D), Wk, bk.reshape(1, D), Wq, bq.reshape(1, D),
      Wv, bv.reshape(1, D), Ws, bs.reshape(1, D))


def _ln(t, w, b):
    m = jnp.mean(t, axis=-1, keepdims=True)
    v = jnp.mean((t - m) ** 2, axis=-1, keepdims=True)
    return (t - m) * lax.rsqrt(v + 1e-5) * w[...] + b[...]


def _head_body(s1, s2, cnt, gnw, gnb, gnm,
               wh0, bh0, lnw0, lnb0, wh1, bh1, lnw1, lnb1, wl, bl, out):
    mg, var, c = _gn_moments(s1[...], s2[...], cnt[...], gnm[...])
    pooled = ((s1[...] - c * mg) * lax.rsqrt(var + 1e-5) * gnw[...] / c
              + gnb[...])
    t = jax.nn.relu(_mm(pooled, wh0[...]) + bh0[...])
    t = _ln(t, lnw0, lnb0)
    t = jax.nn.relu(_mm(t, wh1[...]) + bh1[...])
    t = _ln(t, lnw1, lnb1)
    out[...] = _mm(t, wl[...]) + bl[...]


def _head(s1, s2, cnt, gnw, gnb, gnm,
          Wh0, bh0, lnw0, lnb0, Wh1, bh1, lnw1, lnb1, Wl, bl):
    gsp = pl.BlockSpec((G, D), lambda: (0, 0))

    def vec(n):
        return pl.BlockSpec((1, n), lambda: (0, 0))

    def mat(m, n):
        return pl.BlockSpec((m, n), lambda: (0, 0))

    return pl.pallas_call(
        _head_body,
        in_specs=[gsp, gsp, gsp, vec(D), vec(D), vec(D),
                  mat(D, H1), vec(H1), vec(H1), vec(H1),
                  mat(H1, H2), vec(H2), vec(H2), vec(H2),
                  mat(H2, NCLS), vec(NCLS)],
        out_specs=pl.BlockSpec((G, NCLS), lambda: (0, 0)),
        out_shape=jax.ShapeDtypeStruct((G, NCLS), jnp.float32),
    )(s1, s2, cnt, gnw.reshape(1, D), gnb.reshape(1, D), gnm.reshape(1, D),
      Wh0, bh0.reshape(1, H1), lnw0.reshape(1, H1), lnb0.reshape(1, H1),
      Wh1, bh1.reshape(1, H2), lnw1.reshape(1, H2), lnb1.reshape(1, H2),
      Wl, bl.reshape(1, NCLS))


def kernel(x, edge_index, batch,
           Wk0, bk0, Wq0, bq0, Wv0, bv0, Ws0, bs0, gnw0, gnb0, gnm0,
           Wk1, bk1, Wq1, bq1, Wv1, bv1, Ws1, bs1, gnw1, gnb1, gnm1,
           Wh0, bh0, lnw0, lnb0, Wh1, bh1, lnw1, lnb1, Wl, bl):
    b2d = batch.astype(jnp.int32).reshape(N_NODES, 1)
    ei = edge_index.astype(jnp.int32)
    src, dst = ei[0], ei[1]

    k0, q0, v0, s0 = _proj4(x, Wk0, bk0, Wq0, bq0, Wv0, bv0, Ws0, bs0)
    p = _edge_agg(k0, q0, v0, src, dst)
    hg0, s1, s2, cnt = _stats0(p, s0, b2d)
    k1, q1, v1, s1_ = _normproj(hg0, s1, s2, cnt, b2d, gnw0, gnb0, gnm0,
                                Wk1, bk1, Wq1, bq1, Wv1, bv1, Ws1, bs1)
    p = _edge_agg(k1, q1, v1, src, dst)
    t1, t2, tc = _stats1(p, s1_, b2d)
    return _head(t1, t2, tc, gnw1, gnb1, gnm1,
                 Wh0, bh0, lnw0, lnb0, Wh1, bh1, lnw1, lnb1, Wl, bl)


# speedup vs baseline: 6.1541x; 1.1309x over previous
import functools

import jax
import jax.numpy as jnp
from jax import lax
from jax.experimental import pallas as pl
from jax.experimental.pallas import tpu as pltpu
from jax.experimental.pallas import tpu_sc as plsc

N_NODES = 10000
N_EDGES = 320000
G = 64
D = 128
H1 = 128
H2 = 64
NCLS = 8

NW = 32
EPT = N_EDGES // NW
CHK = 40
NCHUNK = EPT // CHK
N_PAD = 10240
ROWS_PT = N_PAD // 16

BLK = 1000
NBLK = N_NODES // BLK

_HI = lax.Precision.HIGHEST
_C00 = (((0,), (0,)), ((), ()))


def _mm(a, b):
    return jnp.dot(a, b, preferred_element_type=jnp.float32, precision=_HI)


def _gelu(x):
    return x * 0.5 * (1.0 + lax.erf(x * (2.0 ** -0.5)))


def _proj4_body(h_ref, wk, bk, wq, bq, wv, bv, ws, bs, ko, qo, vo, so):
    h = h_ref[...]
    ko[...] = _mm(h, wk[...]) + bk[...]
    qo[...] = _mm(h, wq[...]) + bq[...]
    vo[...] = _mm(h, wv[...]) + bv[...]
    so[...] = _mm(h, ws[...]) + bs[...]


def _proj4(h, Wk, bk, Wq, bq, Wv, bv, Ws, bs):
    blk = 2000
    row = pl.BlockSpec((blk, D), lambda i: (i, 0))
    wsp = pl.BlockSpec((D, D), lambda i: (0, 0))
    bsp = pl.BlockSpec((1, D), lambda i: (0, 0))
    return pl.pallas_call(
        _proj4_body,
        grid=(N_NODES // blk,),
        in_specs=[row, wsp, bsp, wsp, bsp, wsp, bsp, wsp, bsp],
        out_specs=[row, row, row, row],
        out_shape=[jax.ShapeDtypeStruct((N_NODES, D), jnp.float32)] * 4,
    )(h, Wk, bk.reshape(1, D), Wq, bq.reshape(1, D),
      Wv, bv.reshape(1, D), Ws, bs.reshape(1, D))


def _edge_body(k_hbm, q_hbm, v_hbm, src_hbm, dst_hbm, out_hbm,
               is0, id0, kr0, qr0, vr0, is1, id1, kr1, qr1, vr1,
               zbuf, acc, sem0, sem1):
    c = lax.axis_index("c")
    s = lax.axis_index("s")
    wid = c * 16 + s
    base = wid * EPT

    slots = ((is0, id0, kr0, qr0, vr0, sem0),
             (is1, id1, kr1, qr1, vr1, sem1))

    def fetch(i, slot):
        iss, idd, kr, qr, vr, sem = slot
        off = base + i * CHK
        pltpu.sync_copy(src_hbm.at[pl.ds(off, CHK)], iss)
        pltpu.sync_copy(dst_hbm.at[pl.ds(off, CHK)], idd)
        pltpu.async_copy(k_hbm.at[idd], kr, sem)
        pltpu.async_copy(q_hbm.at[iss], qr, sem)
        pltpu.async_copy(v_hbm.at[iss], vr, sem)

    def wait_gathers(slot):
        iss, idd, kr, qr, vr, sem = slot
        pltpu.make_async_copy(k_hbm.at[idd], kr, sem).wait()
        pltpu.make_async_copy(q_hbm.at[iss], qr, sem).wait()
        pltpu.make_async_copy(v_hbm.at[iss], vr, sem).wait()

    def process(slot):
        iss, idd, kr, qr, vr, sem = slot
        wait_gathers(slot)

        @plsc.parallel_loop(0, CHK, 1, unroll=2)
        def _edge(e):
            for j in range(D // 16):
                sl = pl.ds(j * 16, 16)
                kk = kr[e, sl]
                qq = qr[e, sl]
                vv = vr[e, sl]
                vr[e, sl] = vv / (1.0 + jnp.exp(-(kk + qq)))
        pltpu.sync_copy(vr, acc.at[idd], add=True)

    fetch(0, slots[0])

    def _zrow(r, carry):
        for j in range(D // 16):
            zbuf[r, pl.ds(j * 16, 16)] = jnp.zeros((16,), jnp.float32)
        return carry
    lax.fori_loop(0, 64, _zrow, 0)
    for t in range(ROWS_PT // 64):
        pltpu.sync_copy(zbuf, acc.at[pl.ds(s * ROWS_PT + t * 64, 64)])
    plsc.subcore_barrier()

    fetch(1, slots[1])

    def _pair(i2, carry):
        process(slots[0])
        fetch(2 * i2 + 2, slots[0])
        process(slots[1])
        fetch(2 * i2 + 3, slots[1])
        return carry
    lax.fori_loop(0, NCHUNK // 2 - 1, _pair, 0)

    process(slots[0])
    process(slots[1])

    plsc.subcore_barrier()
    for t in range(ROWS_PT // 64):
        r0 = s * ROWS_PT + t * 64
        pltpu.sync_copy(acc.at[pl.ds(r0, 64)], zbuf)
        pltpu.sync_copy(zbuf, out_hbm.at[c, pl.ds(r0, 64)])


_edge_agg = functools.partial(
    pl.kernel,
    out_type=jax.ShapeDtypeStruct((2, N_PAD, D), jnp.float32),
    mesh=plsc.VectorSubcoreMesh(core_axis_name="c", subcore_axis_name="s"),
    scratch_types=[
        pltpu.VMEM((CHK,), jnp.int32),
        pltpu.VMEM((CHK,), jnp.int32),
        pltpu.VMEM((CHK, D), jnp.float32),
        pltpu.VMEM((CHK, D), jnp.float32),
        pltpu.VMEM((CHK, D), jnp.float32),
        pltpu.VMEM((CHK,), jnp.int32),
        pltpu.VMEM((CHK,), jnp.int32),
        pltpu.VMEM((CHK, D), jnp.float32),
        pltpu.VMEM((CHK, D), jnp.float32),
        pltpu.VMEM((CHK, D), jnp.float32),
        pltpu.VMEM((64, D), jnp.float32),
        pltpu.VMEM_SHARED((N_PAD, D), jnp.float32),
        pltpu.SemaphoreType.DMA,
        pltpu.SemaphoreType.DMA,
    ],
)(_edge_body)


def _onehot(b2d, rows):
    P = (b2d[...] == lax.broadcasted_iota(jnp.int32, (rows, G), 1))
    return P.astype(jnp.float32)


def _stats_accum(i, P, hg, s1, s2, cnt):
    @pl.when(i == 0)
    def _():
        s1[...] = jnp.zeros((G, D), jnp.float32)
        s2[...] = jnp.zeros((G, D), jnp.float32)
        cnt[...] = jnp.zeros((G, D), jnp.float32)
    s1[...] += lax.dot_general(P, hg, _C00, precision=_HI)
    s2[...] += lax.dot_general(P, hg * hg, _C00, precision=_HI)
    cnt[...] += lax.dot_general(P, jnp.ones_like(hg), _C00, precision=_HI)


def _stats0_body(p0, p1, sk, b2d, hg_out, s1, s2, cnt):
    i = pl.program_id(0)
    hg = _gelu(p0[0] + p1[0] + sk[...])
    hg_out[...] = hg
    _stats_accum(i, _onehot(b2d, BLK), hg, s1, s2, cnt)


def _stats1_body(p0, p1, sk, b2d, s1, s2, cnt):
    i = pl.program_id(0)
    hg = _gelu(p0[0] + p1[0] + sk[...])
    _stats_accum(i, _onehot(b2d, BLK), hg, s1, s2, cnt)


def _stats_specs():
    prow = lambda core: pl.BlockSpec((1, BLK, D), lambda i, c=core: (c, i, 0))
    row = pl.BlockSpec((BLK, D), lambda i: (i, 0))
    bsp = pl.BlockSpec((BLK, 1), lambda i: (i, 0))
    gsp = pl.BlockSpec((G, D), lambda i: (0, 0))
    return prow, row, bsp, gsp


def _stats0(p, sk, b2d):
    prow, row, bsp, gsp = _stats_specs()
    return pl.pallas_call(
        _stats0_body,
        grid=(NBLK,),
        in_specs=[prow(0), prow(1), row, bsp],
        out_specs=[row, gsp, gsp, gsp],
        out_shape=[jax.ShapeDtypeStruct((N_NODES, D), jnp.float32)]
        + [jax.ShapeDtypeStruct((G, D), jnp.float32)] * 3,
    )(p, p, sk, b2d)


def _stats1(p, sk, b2d):
    prow, row, bsp, gsp = _stats_specs()
    return pl.pallas_call(
        _stats1_body,
        grid=(NBLK,),
        in_specs=[prow(0), prow(1), row, bsp],
        out_specs=[gsp, gsp, gsp],
        out_shape=[jax.ShapeDtypeStruct((G, D), jnp.float32)] * 3,
    )(p, p, sk, b2d)


def _gn_moments(s1, s2, cnt, gnm):
    c = jnp.maximum(cnt, 1.0)
    mg = s1 / c * gnm
    var = (s2 - 2.0 * mg * s1 + c * mg * mg) / c
    return mg, var, c


def _normproj_body(hg, s1, s2, cnt, b2d, gnw, gnb, gnm,
                   wk, bk, wq, bq, wv, bv, ws, bs, ko, qo, vo, so):
    mg, var, _ = _gn_moments(s1[...], s2[...], cnt[...], gnm[...])
    P = _onehot(b2d, BLK)
    o = hg[...] - _mm(P, mg)
    h = o * lax.rsqrt(_mm(P, var) + 1e-5) * gnw[...] + gnb[...]
    ko[...] = _mm(h, wk[...]) + bk[...]
    qo[...] = _mm(h, wq[...]) + bq[...]
    vo[...] = _mm(h, wv[...]) + bv[...]
    so[...] = _mm(h, ws[...]) + bs[...]


def _normproj(hg, s1, s2, cnt, b2d, gnw, gnb, gnm,
              Wk, bk, Wq, bq, Wv, bv, Ws, bs):
    row = pl.BlockSpec((BLK, D), lambda i: (i, 0))
    bsp = pl.BlockSpec((BLK, 1), lambda i: (i, 0))
    gsp = pl.BlockSpec((G, D), lambda i: (0, 0))
    vsp = pl.BlockSpec((1, D), lambda i: (0, 0))
    wsp = pl.BlockSpec((D, D), lambda i: (0, 0))
    return pl.pallas_call(
        _normproj_body,
        grid=(NBLK,),
        in_specs=[row, gsp, gsp, gsp, bsp, vsp, vsp, vsp,
                  wsp, vsp, wsp, vsp, wsp, vsp, wsp, vsp],
        out_specs=[row, row, row, row],
        out_shape=[jax.ShapeDtypeStruct((N_NODES, D), jnp.float32)] * 4,
    )(hg, s1, s2, cnt, b2d, gnw.reshape(1, D), gnb.reshape(1, D),
      gnm.reshape(1, D), Wk, bk.reshape(1, D), Wq, bq.reshape(1, D),
      Wv, bv.reshape(1, D), Ws, bs.reshape(1, D))


def _ln(t, w, b):
    m = jnp.mean(t, axis=-1, keepdims=True)
    v = jnp.mean((t - m) ** 2, axis=-1, keepdims=True)
    return (t - m) * lax.rsqrt(v + 1e-5) * w[...] + b[...]


def _head_body(s1, s2, cnt, gnw, gnb, gnm,
               wh0, bh0, lnw0, lnb0, wh1, bh1, lnw1, lnb1, wl, bl, out):
    mg, var, c = _gn_moments(s1[...], s2[...], cnt[...], gnm[...])
    pooled = ((s1[...] - c * mg) * lax.rsqrt(var + 1e-5) * gnw[...] / c
              + gnb[...])
    t = jax.nn.relu(_mm(pooled, wh0[...]) + bh0[...])
    t = _ln(t, lnw0, lnb0)
    t = jax.nn.relu(_mm(t, wh1[...]) + bh1[...])
    t = _ln(t, lnw1, lnb1)
    out[...] = _mm(t, wl[...]) + bl[...]


def _head(s1, s2, cnt, gnw, gnb, gnm,
          Wh0, bh0, lnw0, lnb0, Wh1, bh1, lnw1, lnb1, Wl, bl):
    gsp = pl.BlockSpec((G, D), lambda: (0, 0))

    def vec(n):
        return pl.BlockSpec((1, n), lambda: (0, 0))

    def mat(m, n):
        return pl.BlockSpec((m, n), lambda: (0, 0))

    return pl.pallas_call(
        _head_body,
        in_specs=[gsp, gsp, gsp, vec(D), vec(D), vec(D),
                  mat(D, H1), vec(H1), vec(H1), vec(H1),
                  mat(H1, H2), vec(H2), vec(H2), vec(H2),
                  mat(H2, NCLS), vec(NCLS)],
        out_specs=pl.BlockSpec((G, NCLS), lambda: (0, 0)),
        out_shape=jax.ShapeDtypeStruct((G, NCLS), jnp.float32),
    )(s1, s2, cnt, gnw.reshape(1, D), gnb.reshape(1, D), gnm.reshape(1, D),
      Wh0, bh0.reshape(1, H1), lnw0.reshape(1, H1), lnb0.reshape(1, H1),
      Wh1, bh1.reshape(1, H2), lnw1.reshape(1, H2), lnb1.reshape(1, H2),
      Wl, bl.reshape(1, NCLS))


def kernel(x, edge_index, batch,
           Wk0, bk0, Wq0, bq0, Wv0, bv0, Ws0, bs0, gnw0, gnb0, gnm0,
           Wk1, bk1, Wq1, bq1, Wv1, bv1, Ws1, bs1, gnw1, gnb1, gnm1,
           Wh0, bh0, lnw0, lnb0, Wh1, bh1, lnw1, lnb1, Wl, bl):
    b2d = batch.astype(jnp.int32).reshape(N_NODES, 1)
    ei = edge_index.astype(jnp.int32)
    src, dst = ei[0], ei[1]

    k0, q0, v0, s0 = _proj4(x, Wk0, bk0, Wq0, bq0, Wv0, bv0, Ws0, bs0)
    p = _edge_agg(k0, q0, v0, src, dst)
    hg0, s1, s2, cnt = _stats0(p, s0, b2d)
    k1, q1, v1, s1_ = _normproj(hg0, s1, s2, cnt, b2d, gnw0, gnb0, gnm0,
                                Wk1, bk1, Wq1, bq1, Wv1, bv1, Ws1, bs1)
    p = _edge_agg(k1, q1, v1, src, dst)
    t1, t2, tc = _stats1(p, s1_, b2d)
    return _head(t1, t2, tc, gnw1, gnb1, gnm1,
                 Wh0, bh0, lnw0, lnb0, Wh1, bh1, lnw1, lnb1, Wl, bl)
